# Initial kernel scaffold; baseline (speedup 1.0000x reference)
import jax
import jax.numpy as jnp
from jax.experimental import pallas as pl


def kernel(hidden_states, router_indices, routing_weights, Wg, bg, Wu, bu, Wd, bd):
    raise NotImplementedError("write your pallas kernel here")


# speedup vs baseline: 1.6041x; 1.6041x over previous
import functools

import jax
import jax.numpy as jnp
from jax import lax
from jax.experimental import pallas as pl
from jax.experimental.pallas import tpu as pltpu
from jax.experimental.pallas import tpu_sc as plsc

ALPHA = 1.702
LIMIT = 7.0

BM = 512
DBLK = 1024
NW = 32
NC = 2


def _mlp_body(bmap_ref, nact_ref, xs_ref, w_ref, wg_ref, bg_ref, wu_ref,
              bu_ref, wd_ref, bd_ref, ys_ref):
    g = pl.program_id(0)
    d = pl.program_id(1)

    @pl.when(d == 0)
    def _init():
        ys_ref[...] = w_ref[...] * bd_ref[0]

    @pl.when(g < nact_ref[0])
    def _accum():
        x = xs_ref[...]
        gate = jnp.dot(x, wg_ref[0], preferred_element_type=jnp.float32)
        up = jnp.dot(x, wu_ref[0], preferred_element_type=jnp.float32)
        gate = jnp.minimum(gate + bg_ref[0], LIMIT)
        up = jnp.clip(up + bu_ref[0], -LIMIT, LIMIT)
        glu = gate * jax.nn.sigmoid(ALPHA * gate)
        act = (up + 1.0) * glu
        y = jnp.dot(act, wd_ref[0], preferred_element_type=jnp.float32)
        ys_ref[...] += w_ref[...] * y


def _grouped_mlp(bmap, nact, xs, w_pad, Wg, bg, Wu, bu, Wd, bd, GG, interpret=False):
    P, H = xs.shape
    D = Wg.shape[2]
    DB = D // DBLK
    grid_spec = pltpu.PrefetchScalarGridSpec(
        num_scalar_prefetch=2,
        grid=(GG, DB),
        in_specs=[
            pl.BlockSpec((BM, H), lambda g, d, bmap, nact: (g, 0)),
            pl.BlockSpec((BM, 1), lambda g, d, bmap, nact: (g, 0)),
            pl.BlockSpec((1, H, DBLK), lambda g, d, bmap, nact: (bmap[g], 0, d)),
            pl.BlockSpec((1, 1, DBLK), lambda g, d, bmap, nact: (bmap[g], 0, d)),
            pl.BlockSpec((1, H, DBLK), lambda g, d, bmap, nact: (bmap[g], 0, d)),
            pl.BlockSpec((1, 1, DBLK), lambda g, d, bmap, nact: (bmap[g], 0, d)),
            pl.BlockSpec((1, DBLK, H), lambda g, d, bmap, nact: (bmap[g], d, 0)),
            pl.BlockSpec((1, 1, H), lambda g, d, bmap, nact: (bmap[g], 0, 0)),
        ],
        out_specs=pl.BlockSpec((BM, H), lambda g, d, bmap, nact: (g, 0)),
    )
    return pl.pallas_call(
        _mlp_body,
        grid_spec=grid_spec,
        out_shape=jax.ShapeDtypeStruct((P, H), jnp.float32),
        compiler_params=pltpu.CompilerParams(
            dimension_semantics=("arbitrary", "arbitrary")),
        interpret=interpret,
    )(bmap, nact, xs, w_pad, Wg, bg.reshape(bg.shape[0], 1, D),
      Wu, bu.reshape(bu.shape[0], 1, D), Wd, bd.reshape(bd.shape[0], 1, H))


def _sc_gather(x, idx3, P, NCH, CH):
    N, H = x.shape
    mesh = plsc.VectorSubcoreMesh(core_axis_name="c", subcore_axis_name="s")

    @functools.partial(
        pl.kernel,
        out_type=jax.ShapeDtypeStruct((P, H), jnp.float32),
        mesh=mesh,
        scratch_types=[
            pltpu.VMEM((NCH, CH), jnp.int32),
            pltpu.VMEM((CH, H), jnp.float32),
            pltpu.SemaphoreType.DMA,
        ],
    )
    def gather_k(x_hbm, idx_hbm, xs_hbm, idx_v, buf, sem):
        wid = lax.axis_index("s") * NC + lax.axis_index("c")
        pltpu.sync_copy(idx_hbm.at[wid], idx_v)
        base = wid * (NCH * CH)
        for c in range(NCH):
            pltpu.async_copy(x_hbm.at[idx_v.at[c]], buf, sem).wait()
            pltpu.sync_copy(buf, xs_hbm.at[pl.ds(base + c * CH, CH)])

    return gather_k(x, idx3)


def _sc_combine(ys, p03, p13, N, NCH, CH):
    H = ys.shape[1]
    mesh = plsc.VectorSubcoreMesh(core_axis_name="c", subcore_axis_name="s")

    @functools.partial(
        pl.kernel,
        out_type=jax.ShapeDtypeStruct((N, H), jnp.float32),
        mesh=mesh,
        scratch_types=[
            pltpu.VMEM((NCH, CH), jnp.int32),
            pltpu.VMEM((NCH, CH), jnp.int32),
            pltpu.VMEM((CH, H), jnp.float32),
            pltpu.VMEM((CH, H), jnp.float32),
            pltpu.SemaphoreType.DMA,
            pltpu.SemaphoreType.DMA,
        ],
    )
    def combine_k(ys_hbm, p0_hbm, p1_hbm, out_hbm, p0_v, p1_v, buf0, buf1,
                  sem0, sem1):
        wid = lax.axis_index("s") * NC + lax.axis_index("c")
        pltpu.sync_copy(p0_hbm.at[wid], p0_v)
        pltpu.sync_copy(p1_hbm.at[wid], p1_v)
        base = wid * (NCH * CH)
        for c in range(NCH):
            cp0 = pltpu.async_copy(ys_hbm.at[p0_v.at[c]], buf0, sem0)
            cp1 = pltpu.async_copy(ys_hbm.at[p1_v.at[c]], buf1, sem1)
            cp0.wait()
            cp1.wait()
            for r in range(CH):
                def add_j(j, _, r=r):
                    jj = pl.multiple_of(j * 16, 16)
                    buf0[r, pl.ds(jj, 16)] = (buf0[r, pl.ds(jj, 16)]
                                              + buf1[r, pl.ds(jj, 16)])
                    return 0
                lax.fori_loop(0, H // 16, add_j, 0)
            pltpu.sync_copy(buf0, out_hbm.at[pl.ds(base + c * CH, CH)])

    return combine_k(ys, p03, p13)


def _dispatch_metadata(router_indices, routing_weights, N, E, K, GG):
    S = N * K
    P = GG * BM
    eid = router_indices.reshape(S).astype(jnp.int32)
    tok = jnp.arange(S, dtype=jnp.int32) // K
    onehot = (eid[:, None] == jnp.arange(E, dtype=jnp.int32)[None, :]
              ).astype(jnp.int32)
    ranks = jnp.cumsum(onehot, axis=0) - onehot
    rank = jnp.take_along_axis(ranks, eid[:, None], axis=1)[:, 0]
    counts = jnp.sum(onehot, axis=0)
    nblk = (counts + BM - 1) // BM
    cum = jnp.cumsum(nblk)
    blkstart = cum - nblk
    p = blkstart[eid] * BM + rank
    nact = cum[-1].astype(jnp.int32).reshape(1)
    bmap = jnp.searchsorted(cum, jnp.arange(GG, dtype=jnp.int32),
                            side='right').astype(jnp.int32)
    bmap = jnp.minimum(bmap, E - 1)
    row_map = jnp.zeros(P, jnp.int32).at[p].set(tok)
    rw_flat = routing_weights.reshape(N, E)
    w_slot = rw_flat[tok, eid]
    w_pad = jnp.zeros(P, jnp.float32).at[p].set(w_slot).reshape(P, 1)
    pos = p.reshape(N, K)
    return bmap, nact, row_map, w_pad, pos


def kernel(hidden_states, router_indices, routing_weights, Wg, bg, Wu, bu,
           Wd, bd):
    B, T, H = hidden_states.shape
    E, _, D = Wg.shape
    K = router_indices.shape[-1]
    N = B * T
    S = N * K
    GG = -(-S // BM) + E - 1
    GG = ((GG + 7) // 8) * 8
    P = GG * BM

    bmap, nact, row_map, w_pad, pos = _dispatch_metadata(
        router_indices, routing_weights, N, E, K, GG)

    x = hidden_states.reshape(N, H)

    RPW = P // NW
    CH_G = 48 if RPW % 48 == 0 else 32
    NCH_G = RPW // CH_G
    idx3 = row_map.reshape(NW, NCH_G, CH_G)
    xs = _sc_gather(x, idx3, P, NCH_G, CH_G)

    ys = _grouped_mlp(bmap, nact, xs, w_pad, Wg, bg, Wu, bu, Wd, bd, GG)

    TPW = N // NW
    CH_C = 32
    NCH_C = TPW // CH_C
    p03 = pos[:, 0].reshape(NW, NCH_C, CH_C)
    p13 = pos[:, 1].reshape(NW, NCH_C, CH_C)
    out = _sc_combine(ys, p03, p13, N, NCH_C, CH_C)

    return out.reshape(B, T, H)
